# Initial kernel scaffold; baseline (speedup 1.0000x reference)
import jax
import jax.numpy as jnp
from jax.experimental import pallas as pl


def kernel(x, edge_index, W_neigh, W_self, b):
    raise NotImplementedError("write your pallas kernel here")


# speedup vs baseline: 2.7731x; 2.7731x over previous
import functools

import jax
import jax.numpy as jnp
from jax import lax
from jax.experimental import pallas as pl
from jax.experimental.pallas import tpu as pltpu
from jax.experimental.pallas import tpu_sc as plsc

NC = 2
NS = 16
NW = NC * NS
L = 16
CH = 128
BLKC = 16


def _agg_call(N, D, NCH):
    n_acc = -(-(N + 1) // 128) * 128
    rows_per = n_acc // NS
    mesh = plsc.VectorSubcoreMesh(core_axis_name="c", subcore_axis_name="s")

    @functools.partial(
        pl.kernel,
        out_type=jax.ShapeDtypeStruct((NC, n_acc, D), jnp.float32),
        mesh=mesh,
        scratch_types=[
            pltpu.VMEM((BLKC, CH), jnp.int32),
            pltpu.VMEM((BLKC, CH), jnp.int32),
            pltpu.VMEM((CH, D), jnp.float32),
            pltpu.VMEM((CH, D), jnp.float32),
            pltpu.VMEM_SHARED((n_acc, D), jnp.float32),
            pltpu.SemaphoreType.DMA,
            pltpu.SemaphoreType.DMA,
        ],
    )
    def agg(x_hbm, src_hbm, tgt_hbm, zero_hbm, part_hbm,
            src_v, tgt_v, buf0, buf1, acc, sem0, sem1):
        c = lax.axis_index("c")
        s = lax.axis_index("s")
        base = (c * NS + s) * NCH
        bufs, sems = (buf0, buf1), (sem0, sem1)
        r0 = s * rows_per
        pltpu.sync_copy(zero_hbm.at[pl.ds(r0, rows_per)],
                        acc.at[pl.ds(r0, rows_per)])
        plsc.subcore_barrier()

        @pl.loop(0, NCH // BLKC)
        def _(ib):
            b0 = base + ib * BLKC
            pltpu.sync_copy(src_hbm.at[pl.ds(b0, BLKC)], src_v)
            pltpu.sync_copy(tgt_hbm.at[pl.ds(b0, BLKC)], tgt_v)
            pltpu.async_copy(x_hbm.at[src_v.at[0]], buf0, sem0)
            pltpu.async_copy(x_hbm.at[src_v.at[1]], buf1, sem1)

            @pl.loop(0, BLKC // 2 - 1)
            def _(i):
                for bi in range(2):
                    j = i * 2 + bi
                    pltpu.make_async_copy(x_hbm.at[src_v.at[j]], bufs[bi],
                                          sems[bi]).wait()
                    pltpu.sync_copy(bufs[bi], acc.at[tgt_v.at[j]], add=True)
                    pltpu.async_copy(x_hbm.at[src_v.at[j + 2]], bufs[bi],
                                     sems[bi])

            for bi in range(2):
                j = BLKC - 2 + bi
                pltpu.make_async_copy(x_hbm.at[src_v.at[j]], bufs[bi],
                                      sems[bi]).wait()
                pltpu.sync_copy(bufs[bi], acc.at[tgt_v.at[j]], add=True)

        plsc.subcore_barrier()
        pltpu.sync_copy(acc.at[pl.ds(r0, rows_per)],
                        part_hbm.at[c].at[pl.ds(r0, rows_per)])

    return agg


def _dot_call(N, D, NCH):
    mesh = plsc.VectorSubcoreMesh(core_axis_name="c", subcore_axis_name="s")

    @functools.partial(
        pl.kernel,
        out_type=jax.ShapeDtypeStruct((NW * NCH * CH, L), jnp.float32),
        mesh=mesh,
        scratch_types=[
            pltpu.VMEM((NCH, CH), jnp.int32),
            pltpu.VMEM((NCH, CH), jnp.int32),
            pltpu.VMEM((CH, D), jnp.float32),
            pltpu.VMEM((CH, D), jnp.float32),
            pltpu.VMEM((CH, D), jnp.float32),
            pltpu.VMEM((CH, D), jnp.float32),
            pltpu.VMEM((CH, L), jnp.float32),
            pltpu.VMEM((CH, L), jnp.float32),
            pltpu.SemaphoreType.DMA,
            pltpu.SemaphoreType.DMA,
            pltpu.SemaphoreType.DMA,
            pltpu.SemaphoreType.DMA,
        ],
    )
    def dot(h_hbm, src_hbm, tgt_hbm, out_hbm,
            src_v, tgt_v, bs0, bt0, bs1, bt1, pa0, pa1,
            sem0, sem1, semo0, semo1):
        c = lax.axis_index("c")
        s = lax.axis_index("s")
        base = (c * NS + s) * NCH
        pltpu.sync_copy(src_hbm.at[pl.ds(base, NCH)], src_v)
        pltpu.sync_copy(tgt_hbm.at[pl.ds(base, NCH)], tgt_v)
        bs, bt = (bs0, bs1), (bt0, bt1)
        pa = (pa0, pa1)
        sems = (sem0, sem1)
        semo = (semo0, semo1)

        def out_rows(j):
            return out_hbm.at[pl.ds((base + j) * CH, CH)]

        def compute(j, bi):
            pltpu.make_async_copy(h_hbm.at[src_v.at[j]], bs[bi],
                                  sems[bi]).wait()
            pltpu.make_async_copy(h_hbm.at[tgt_v.at[j]], bt[bi],
                                  sems[bi]).wait()

            @pl.loop(0, CH, unroll=2)
            def _(e):
                acc = bs[bi][e, pl.ds(0, L)] * bt[bi][e, pl.ds(0, L)]
                for k in range(1, D // L):
                    acc = acc + (bs[bi][e, pl.ds(k * L, L)] *
                                 bt[bi][e, pl.ds(k * L, L)])
                pa[bi][e, :] = acc

        for bi in range(2):
            pltpu.async_copy(h_hbm.at[src_v.at[bi]], bs[bi], sems[bi])
            pltpu.async_copy(h_hbm.at[tgt_v.at[bi]], bt[bi], sems[bi])
        for bi in range(2):
            compute(bi, bi)
            pltpu.async_copy(h_hbm.at[src_v.at[bi + 2]], bs[bi], sems[bi])
            pltpu.async_copy(h_hbm.at[tgt_v.at[bi + 2]], bt[bi], sems[bi])
            pltpu.async_copy(pa[bi], out_rows(bi), semo[bi])

        @pl.loop(1, NCH // 2 - 1)
        def _(i):
            for bi in range(2):
                j = i * 2 + bi
                pltpu.make_async_copy(pa[bi], out_rows(j), semo[bi]).wait()
                compute(j, bi)
                pltpu.async_copy(h_hbm.at[src_v.at[j + 2]], bs[bi], sems[bi])
                pltpu.async_copy(h_hbm.at[tgt_v.at[j + 2]], bt[bi], sems[bi])
                pltpu.async_copy(pa[bi], out_rows(j), semo[bi])

        for bi in range(2):
            j = NCH - 2 + bi
            pltpu.make_async_copy(pa[bi], out_rows(j), semo[bi]).wait()
            compute(j, bi)
            pltpu.async_copy(pa[bi], out_rows(j), semo[bi])
        for bi in range(2):
            pltpu.make_async_copy(pa[bi], out_rows(0), semo[bi]).wait()

    return dot


def _reduce16(p):
    M = p.shape[0]
    BLK = 8192

    def red(pr, outr):
        s = jnp.sum(pr[...], axis=1)
        outr[...] = s.reshape(BLK // 128, 128)

    return pl.pallas_call(
        red,
        grid=(M // BLK,),
        in_specs=[pl.BlockSpec((BLK, L), lambda i: (i, 0))],
        out_specs=pl.BlockSpec((BLK // 128, 128), lambda i: (i, 0)),
        out_shape=jax.ShapeDtypeStruct((M // 128, 128), jnp.float32),
    )(p)


def _dense(p0, p1, x, W_neigh, W_self, b2):
    N, D = x.shape
    BLK = 2000

    def mm(p0r, p1r, xr, wn, ws, br, hr):
        agg = p0r[...] + p1r[...]
        acc = jnp.dot(agg, wn[...], preferred_element_type=jnp.float32)
        acc = acc + jnp.dot(xr[...], ws[...], preferred_element_type=jnp.float32)
        hr[...] = jnp.maximum(acc + br[...], 0.0)

    row_spec = pl.BlockSpec((BLK, D), lambda i: (i, 0))
    w_spec = pl.BlockSpec((D, D), lambda i: (0, 0))
    return pl.pallas_call(
        mm,
        grid=(N // BLK,),
        in_specs=[row_spec, row_spec, row_spec, w_spec, w_spec,
                  pl.BlockSpec((1, D), lambda i: (0, 0))],
        out_specs=row_spec,
        out_shape=jax.ShapeDtypeStruct((N, D), jnp.float32),
    )(p0, p1, x, W_neigh, W_self, b2)


def kernel(x, edge_index, W_neigh, W_self, b):
    N, D = x.shape
    E = edge_index.shape[1]
    n_acc = -(-(N + 1) // 128) * 128
    NCH = -(-E // (CH * NW))
    NCH = -(-NCH // BLKC) * BLKC
    e_pad = NW * NCH * CH
    pad = e_pad - E

    src = edge_index[0]
    tgt = edge_index[1]
    src_p = jnp.concatenate(
        [src, jnp.zeros((pad,), jnp.int32)]).reshape(NW * NCH, CH)
    tgt_a = jnp.concatenate(
        [tgt, jnp.full((pad,), N, jnp.int32)]).reshape(NW * NCH, CH)
    tgt_c = jnp.concatenate(
        [tgt, jnp.zeros((pad,), jnp.int32)]).reshape(NW * NCH, CH)
    zeros = jnp.zeros((n_acc, D), jnp.float32)

    parts = _agg_call(N, D, NCH)(x, src_p, tgt_a, zeros)
    h = _dense(parts[0, :N], parts[1, :N], x, W_neigh, W_self,
               b.reshape(1, D))
    partial16 = _dot_call(N, D, NCH)(h, src_p, tgt_c)
    scores = _reduce16(partial16)
    return scores.reshape(-1)[:E]


# speedup vs baseline: 2.8038x; 1.0111x over previous
import functools

import jax
import jax.numpy as jnp
from jax import lax
from jax.experimental import pallas as pl
from jax.experimental.pallas import tpu as pltpu
from jax.experimental.pallas import tpu_sc as plsc

NC = 2
NS = 16
NW = NC * NS
L = 16
CH = 128
BLKC = 16


def _agg_call(N, D, NCH):
    n_acc = -(-(N + 1) // 128) * 128
    rows_per = n_acc // NS
    mesh = plsc.VectorSubcoreMesh(core_axis_name="c", subcore_axis_name="s")

    @functools.partial(
        pl.kernel,
        out_type=jax.ShapeDtypeStruct((NC, n_acc, D), jnp.float32),
        mesh=mesh,
        scratch_types=[
            pltpu.VMEM((BLKC, CH), jnp.int32),
            pltpu.VMEM((BLKC, CH), jnp.int32),
            pltpu.VMEM((CH, D), jnp.float32),
            pltpu.VMEM((CH, D), jnp.float32),
            pltpu.VMEM_SHARED((n_acc, D), jnp.float32),
            pltpu.SemaphoreType.DMA,
            pltpu.SemaphoreType.DMA,
        ],
    )
    def agg(x_hbm, src_hbm, tgt_hbm, zero_hbm, part_hbm,
            src_v, tgt_v, buf0, buf1, acc, sem0, sem1):
        c = lax.axis_index("c")
        s = lax.axis_index("s")
        base = (c * NS + s) * NCH
        bufs, sems = (buf0, buf1), (sem0, sem1)
        r0 = s * rows_per
        pltpu.sync_copy(zero_hbm.at[pl.ds(r0, rows_per)],
                        acc.at[pl.ds(r0, rows_per)])
        plsc.subcore_barrier()

        @pl.loop(0, NCH // BLKC)
        def _(ib):
            b0 = base + ib * BLKC
            pltpu.sync_copy(src_hbm.at[pl.ds(b0, BLKC)], src_v)
            pltpu.sync_copy(tgt_hbm.at[pl.ds(b0, BLKC)], tgt_v)
            pltpu.async_copy(x_hbm.at[src_v.at[0]], buf0, sem0)
            pltpu.async_copy(x_hbm.at[src_v.at[1]], buf1, sem1)

            @pl.loop(0, BLKC // 2 - 1)
            def _(i):
                for bi in range(2):
                    j = i * 2 + bi
                    pltpu.make_async_copy(x_hbm.at[src_v.at[j]], bufs[bi],
                                          sems[bi]).wait()
                    pltpu.sync_copy(bufs[bi], acc.at[tgt_v.at[j]], add=True)
                    pltpu.async_copy(x_hbm.at[src_v.at[j + 2]], bufs[bi],
                                     sems[bi])

            for bi in range(2):
                j = BLKC - 2 + bi
                pltpu.make_async_copy(x_hbm.at[src_v.at[j]], bufs[bi],
                                      sems[bi]).wait()
                pltpu.sync_copy(bufs[bi], acc.at[tgt_v.at[j]], add=True)

        plsc.subcore_barrier()
        pltpu.sync_copy(acc.at[pl.ds(r0, rows_per)],
                        part_hbm.at[c].at[pl.ds(r0, rows_per)])

    return agg


def _dot_call(N, D, NCH):
    mesh = plsc.VectorSubcoreMesh(core_axis_name="c", subcore_axis_name="s")

    @functools.partial(
        pl.kernel,
        out_type=jax.ShapeDtypeStruct((NW * NCH * CH * L,), jnp.float32),
        mesh=mesh,
        scratch_types=[
            pltpu.VMEM((NCH, CH), jnp.int32),
            pltpu.VMEM((NCH, CH), jnp.int32),
            pltpu.VMEM((CH, D), jnp.float32),
            pltpu.VMEM((CH, D), jnp.float32),
            pltpu.VMEM((CH, D), jnp.float32),
            pltpu.VMEM((CH, D), jnp.float32),
            pltpu.VMEM((CH * L,), jnp.float32),
            pltpu.VMEM((CH * L,), jnp.float32),
            pltpu.SemaphoreType.DMA,
            pltpu.SemaphoreType.DMA,
            pltpu.SemaphoreType.DMA,
            pltpu.SemaphoreType.DMA,
        ],
    )
    def dot(h_hbm, src_hbm, tgt_hbm, out_hbm,
            src_v, tgt_v, bs0, bt0, bs1, bt1, pa0, pa1,
            sem0, sem1, semo0, semo1):
        c = lax.axis_index("c")
        s = lax.axis_index("s")
        base = (c * NS + s) * NCH
        pltpu.sync_copy(src_hbm.at[pl.ds(base, NCH)], src_v)
        pltpu.sync_copy(tgt_hbm.at[pl.ds(base, NCH)], tgt_v)
        bs, bt = (bs0, bs1), (bt0, bt1)
        pa = (pa0, pa1)
        sems = (sem0, sem1)
        semo = (semo0, semo1)

        def out_rows(j):
            return out_hbm.at[pl.ds((base + j) * CH * L, CH * L)]

        def compute(j, bi):
            pltpu.make_async_copy(h_hbm.at[src_v.at[j]], bs[bi],
                                  sems[bi]).wait()
            pltpu.make_async_copy(h_hbm.at[tgt_v.at[j]], bt[bi],
                                  sems[bi]).wait()

            @pl.loop(0, CH, unroll=2)
            def _(e):
                acc = bs[bi][e, pl.ds(0, L)] * bt[bi][e, pl.ds(0, L)]
                for k in range(1, D // L):
                    acc = acc + (bs[bi][e, pl.ds(k * L, L)] *
                                 bt[bi][e, pl.ds(k * L, L)])
                pa[bi][pl.ds(e * L, L)] = acc

        for bi in range(2):
            pltpu.async_copy(h_hbm.at[src_v.at[bi]], bs[bi], sems[bi])
            pltpu.async_copy(h_hbm.at[tgt_v.at[bi]], bt[bi], sems[bi])
        for bi in range(2):
            compute(bi, bi)
            pltpu.async_copy(h_hbm.at[src_v.at[bi + 2]], bs[bi], sems[bi])
            pltpu.async_copy(h_hbm.at[tgt_v.at[bi + 2]], bt[bi], sems[bi])
            pltpu.async_copy(pa[bi], out_rows(bi), semo[bi])

        @pl.loop(1, NCH // 2 - 1)
        def _(i):
            for bi in range(2):
                j = i * 2 + bi
                pltpu.make_async_copy(pa[bi], out_rows(j), semo[bi]).wait()
                compute(j, bi)
                pltpu.async_copy(h_hbm.at[src_v.at[j + 2]], bs[bi], sems[bi])
                pltpu.async_copy(h_hbm.at[tgt_v.at[j + 2]], bt[bi], sems[bi])
                pltpu.async_copy(pa[bi], out_rows(j), semo[bi])

        for bi in range(2):
            j = NCH - 2 + bi
            pltpu.make_async_copy(pa[bi], out_rows(j), semo[bi]).wait()
            compute(j, bi)
            pltpu.async_copy(pa[bi], out_rows(j), semo[bi])
        for bi in range(2):
            pltpu.make_async_copy(pa[bi], out_rows(0), semo[bi]).wait()

    return dot


def _reduce16(p):
    M = p.shape[0] // 128
    BLK = 4096
    EPR = 128 // L

    def red(pr, outr):
        x = pr[...]
        cols = [jnp.sum(x[:, e * L:(e + 1) * L], axis=1, keepdims=True)
                for e in range(EPR)]
        outr[...] = jnp.concatenate(cols, axis=1)

    return pl.pallas_call(
        red,
        grid=(M // BLK,),
        in_specs=[pl.BlockSpec((BLK, 128), lambda i: (i, 0))],
        out_specs=pl.BlockSpec((BLK, EPR), lambda i: (i, 0)),
        out_shape=jax.ShapeDtypeStruct((M, EPR), jnp.float32),
    )(p.reshape(M, 128))


def _dense(p0, p1, x, W_neigh, W_self, b2):
    N, D = x.shape
    BLK = 2000

    def mm(p0r, p1r, xr, wn, ws, br, hr):
        agg = p0r[...] + p1r[...]
        acc = jnp.dot(agg, wn[...], preferred_element_type=jnp.float32)
        acc = acc + jnp.dot(xr[...], ws[...], preferred_element_type=jnp.float32)
        hr[...] = jnp.maximum(acc + br[...], 0.0)

    row_spec = pl.BlockSpec((BLK, D), lambda i: (i, 0))
    w_spec = pl.BlockSpec((D, D), lambda i: (0, 0))
    return pl.pallas_call(
        mm,
        grid=(N // BLK,),
        in_specs=[row_spec, row_spec, row_spec, w_spec, w_spec,
                  pl.BlockSpec((1, D), lambda i: (0, 0))],
        out_specs=row_spec,
        out_shape=jax.ShapeDtypeStruct((N, D), jnp.float32),
    )(p0, p1, x, W_neigh, W_self, b2)


def kernel(x, edge_index, W_neigh, W_self, b):
    N, D = x.shape
    E = edge_index.shape[1]
    n_acc = -(-(N + 1) // 128) * 128
    NCH = -(-E // (CH * NW))
    NCH = -(-NCH // BLKC) * BLKC
    e_pad = NW * NCH * CH
    pad = e_pad - E

    src = edge_index[0]
    tgt = edge_index[1]
    src_p = jnp.concatenate(
        [src, jnp.zeros((pad,), jnp.int32)]).reshape(NW * NCH, CH)
    tgt_a = jnp.concatenate(
        [tgt, jnp.full((pad,), N, jnp.int32)]).reshape(NW * NCH, CH)
    tgt_c = jnp.concatenate(
        [tgt, jnp.zeros((pad,), jnp.int32)]).reshape(NW * NCH, CH)
    zeros = jnp.zeros((n_acc, D), jnp.float32)

    parts = _agg_call(N, D, NCH)(x, src_p, tgt_a, zeros)
    h = _dense(parts[0, :N], parts[1, :N], x, W_neigh, W_self,
               b.reshape(1, D))
    partial16 = _dot_call(N, D, NCH)(h, src_p, tgt_c)
    scores = _reduce16(partial16)
    return scores.reshape(-1)[:E]


# speedup vs baseline: 4.9092x; 1.7509x over previous
import functools

import jax
import jax.numpy as jnp
from jax import lax
from jax.experimental import pallas as pl
from jax.experimental.pallas import tpu as pltpu
from jax.experimental.pallas import tpu_sc as plsc

NC = 2
NS = 16
NW = NC * NS
L = 16
CH = 128
BLKC = 16


def _agg_call(N, D, NCH):
    n_acc = -(-(N + 1) // 128) * 128
    rows_per = n_acc // NS
    mesh = plsc.VectorSubcoreMesh(core_axis_name="c", subcore_axis_name="s")

    @functools.partial(
        pl.kernel,
        out_type=jax.ShapeDtypeStruct((NC, n_acc, D), jnp.float32),
        mesh=mesh,
        scratch_types=[
            pltpu.VMEM((BLKC, CH), jnp.int32),
            pltpu.VMEM((BLKC, CH), jnp.int32),
            pltpu.VMEM((CH, D), jnp.float32),
            pltpu.VMEM((CH, D), jnp.float32),
            pltpu.VMEM_SHARED((n_acc, D), jnp.float32),
            pltpu.SemaphoreType.DMA,
            pltpu.SemaphoreType.DMA,
        ],
    )
    def agg(x_hbm, src_hbm, tgt_hbm, zero_hbm, part_hbm,
            src_v, tgt_v, buf0, buf1, acc, sem0, sem1):
        c = lax.axis_index("c")
        s = lax.axis_index("s")
        base = (c * NS + s) * NCH
        bufs, sems = (buf0, buf1), (sem0, sem1)
        r0 = s * rows_per
        pltpu.sync_copy(zero_hbm.at[pl.ds(r0, rows_per)],
                        acc.at[pl.ds(r0, rows_per)])
        plsc.subcore_barrier()

        @pl.loop(0, NCH // BLKC)
        def _(ib):
            b0 = base + ib * BLKC
            pltpu.sync_copy(src_hbm.at[pl.ds(b0, BLKC)], src_v)
            pltpu.sync_copy(tgt_hbm.at[pl.ds(b0, BLKC)], tgt_v)
            pltpu.async_copy(x_hbm.at[src_v.at[0]], buf0, sem0)
            pltpu.async_copy(x_hbm.at[src_v.at[1]], buf1, sem1)

            @pl.loop(0, BLKC // 2 - 1)
            def _(i):
                for bi in range(2):
                    j = i * 2 + bi
                    pltpu.make_async_copy(x_hbm.at[src_v.at[j]], bufs[bi],
                                          sems[bi]).wait()
                    pltpu.sync_copy(bufs[bi], acc.at[tgt_v.at[j]], add=True)
                    pltpu.async_copy(x_hbm.at[src_v.at[j + 2]], bufs[bi],
                                     sems[bi])

            for bi in range(2):
                j = BLKC - 2 + bi
                pltpu.make_async_copy(x_hbm.at[src_v.at[j]], bufs[bi],
                                      sems[bi]).wait()
                pltpu.sync_copy(bufs[bi], acc.at[tgt_v.at[j]], add=True)

        plsc.subcore_barrier()
        pltpu.sync_copy(acc.at[pl.ds(r0, rows_per)],
                        part_hbm.at[c].at[pl.ds(r0, rows_per)])

    return agg


def _dot_call(N, D, NCH, NP):
    CE = CH // 2
    B2 = 2 * BLKC
    NB = NCH // BLKC
    rows_stage = NP // NS
    mesh = plsc.VectorSubcoreMesh(core_axis_name="c", subcore_axis_name="s")

    @functools.partial(
        pl.kernel,
        out_type=jax.ShapeDtypeStruct((NW * NCH * CH * L,), jnp.float32),
        mesh=mesh,
        scratch_types=[
            pltpu.VMEM((BLKC, CH), jnp.int32),
            pltpu.VMEM((BLKC, CH), jnp.int32),
            pltpu.VMEM_SHARED((NP, D), jnp.float32),
            pltpu.VMEM((CE, D), jnp.float32),
            pltpu.VMEM((CE, D), jnp.float32),
            pltpu.VMEM((CE, D), jnp.float32),
            pltpu.VMEM((CE, D), jnp.float32),
            pltpu.VMEM((CE * L,), jnp.float32),
            pltpu.VMEM((CE * L,), jnp.float32),
            pltpu.SemaphoreType.DMA,
            pltpu.SemaphoreType.DMA,
            pltpu.SemaphoreType.DMA,
            pltpu.SemaphoreType.DMA,
        ],
    )
    def dot(h_hbm, src_hbm, tgt_hbm, out_hbm,
            src_v, tgt_v, hsp, bs0, bt0, bs1, bt1, pa0, pa1,
            sem0, sem1, semo0, semo1):
        c = lax.axis_index("c")
        s = lax.axis_index("s")
        w = c * NS + s
        bs, bt = (bs0, bs1), (bt0, bt1)
        pa = (pa0, pa1)
        sems = (sem0, sem1)
        semo = (semo0, semo1)

        r0 = s * rows_stage
        pltpu.sync_copy(h_hbm.at[pl.ds(r0, rows_stage)],
                        hsp.at[pl.ds(r0, rows_stage)])
        plsc.subcore_barrier()

        def idx_ref(v, hh):
            return v.at[hh // 2, pl.ds((hh % 2) * CE, CE)]

        def out_ref(ib, hh):
            off = (w * NCH * CH + ib * BLKC * CH + hh * CE) * L
            return out_hbm.at[pl.ds(off, CE * L)]

        def fire(hh, bi):
            pltpu.async_copy(hsp.at[idx_ref(src_v, hh)], bs[bi], sems[bi])
            pltpu.async_copy(hsp.at[idx_ref(tgt_v, hh)], bt[bi], sems[bi])

        def compute(hh, bi):
            pltpu.make_async_copy(hsp.at[idx_ref(src_v, hh)], bs[bi],
                                  sems[bi]).wait()
            pltpu.make_async_copy(hsp.at[idx_ref(tgt_v, hh)], bt[bi],
                                  sems[bi]).wait()

            @pl.loop(0, CE, unroll=4)
            def _(e):
                acc = bs[bi][e, pl.ds(0, L)] * bt[bi][e, pl.ds(0, L)]
                for k in range(1, D // L):
                    acc = acc + (bs[bi][e, pl.ds(k * L, L)] *
                                 bt[bi][e, pl.ds(k * L, L)])
                pa[bi][pl.ds(e * L, L)] = acc

        @pl.loop(0, NB)
        def _(ib):
            b0 = (w * NCH) + ib * BLKC
            pltpu.sync_copy(src_hbm.at[pl.ds(b0, BLKC)], src_v)
            pltpu.sync_copy(tgt_hbm.at[pl.ds(b0, BLKC)], tgt_v)
            for bi in range(2):
                fire(bi, bi)
            for bi in range(2):
                compute(bi, bi)
                fire(bi + 2, bi)
                pltpu.async_copy(pa[bi], out_ref(ib, bi), semo[bi])

            @pl.loop(1, B2 // 2 - 1)
            def _(i):
                for bi in range(2):
                    hh = i * 2 + bi
                    pltpu.make_async_copy(pa[bi], out_ref(ib, hh),
                                          semo[bi]).wait()
                    compute(hh, bi)
                    fire(hh + 2, bi)
                    pltpu.async_copy(pa[bi], out_ref(ib, hh), semo[bi])

            for bi in range(2):
                hh = B2 - 2 + bi
                pltpu.make_async_copy(pa[bi], out_ref(ib, hh),
                                      semo[bi]).wait()
                compute(hh, bi)
                pltpu.async_copy(pa[bi], out_ref(ib, hh), semo[bi])
            for bi in range(2):
                pltpu.make_async_copy(pa[bi], out_ref(ib, 0),
                                      semo[bi]).wait()

    return dot


def _reduce16(p):
    M = p.shape[0] // 128
    BLK = 4096
    EPR = 128 // L

    def red(pr, outr):
        x = pr[...]
        cols = [jnp.sum(x[:, e * L:(e + 1) * L], axis=1, keepdims=True)
                for e in range(EPR)]
        outr[...] = jnp.concatenate(cols, axis=1)

    return pl.pallas_call(
        red,
        grid=(M // BLK,),
        in_specs=[pl.BlockSpec((BLK, 128), lambda i: (i, 0))],
        out_specs=pl.BlockSpec((BLK, EPR), lambda i: (i, 0)),
        out_shape=jax.ShapeDtypeStruct((M, EPR), jnp.float32),
    )(p.reshape(M, 128))


def _dense(p0, p1, x, W_neigh, W_self, b2):
    N, D = x.shape
    BLK = 2000

    def mm(p0r, p1r, xr, wn, ws, br, hr):
        agg = p0r[...] + p1r[...]
        acc = jnp.dot(agg, wn[...], preferred_element_type=jnp.float32)
        acc = acc + jnp.dot(xr[...], ws[...], preferred_element_type=jnp.float32)
        hr[...] = jnp.maximum(acc + br[...], 0.0)

    row_spec = pl.BlockSpec((BLK, D), lambda i: (i, 0))
    w_spec = pl.BlockSpec((D, D), lambda i: (0, 0))
    return pl.pallas_call(
        mm,
        grid=(N // BLK,),
        in_specs=[row_spec, row_spec, row_spec, w_spec, w_spec,
                  pl.BlockSpec((1, D), lambda i: (0, 0))],
        out_specs=row_spec,
        out_shape=jax.ShapeDtypeStruct((N, D), jnp.float32),
    )(p0, p1, x, W_neigh, W_self, b2)


def kernel(x, edge_index, W_neigh, W_self, b):
    N, D = x.shape
    E = edge_index.shape[1]
    n_acc = -(-(N + 1) // 128) * 128
    NCH = -(-E // (CH * NW))
    NCH = -(-NCH // BLKC) * BLKC
    e_pad = NW * NCH * CH
    pad = e_pad - E

    src = edge_index[0]
    tgt = edge_index[1]
    src_p = jnp.concatenate(
        [src, jnp.zeros((pad,), jnp.int32)]).reshape(NW * NCH, CH)
    tgt_a = jnp.concatenate(
        [tgt, jnp.full((pad,), N, jnp.int32)]).reshape(NW * NCH, CH)
    tgt_c = jnp.concatenate(
        [tgt, jnp.zeros((pad,), jnp.int32)]).reshape(NW * NCH, CH)
    zeros = jnp.zeros((n_acc, D), jnp.float32)

    parts = _agg_call(N, D, NCH)(x, src_p, tgt_a, zeros)
    h = _dense(parts[0, :N], parts[1, :N], x, W_neigh, W_self,
               b.reshape(1, D))
    h_pad = jnp.concatenate([h, jnp.zeros((n_acc - N, D), jnp.float32)])
    partial16 = _dot_call(N, D, NCH, n_acc)(h_pad, src_p, tgt_c)
    scores = _reduce16(partial16)
    return scores.reshape(-1)[:E]


# speedup vs baseline: 4.9182x; 1.0018x over previous
import functools

import jax
import jax.numpy as jnp
from jax import lax
from jax.experimental import pallas as pl
from jax.experimental.pallas import tpu as pltpu
from jax.experimental.pallas import tpu_sc as plsc

NC = 2
NS = 16
NW = NC * NS
L = 16
CH = 128
BLKC = 16


def _agg_call(N, D, NCH):
    n_acc = -(-(N + 1) // 128) * 128
    rows_per = n_acc // NS
    mesh = plsc.VectorSubcoreMesh(core_axis_name="c", subcore_axis_name="s")

    CE = CH // 2
    B2 = 2 * BLKC
    NSLOT = 4

    @functools.partial(
        pl.kernel,
        out_type=jax.ShapeDtypeStruct((NC, n_acc, D), jnp.float32),
        mesh=mesh,
        scratch_types=[
            pltpu.VMEM((B2, CE), jnp.int32),
            pltpu.VMEM((B2, CE), jnp.int32),
            pltpu.VMEM((CE, D), jnp.float32),
            pltpu.VMEM((CE, D), jnp.float32),
            pltpu.VMEM((CE, D), jnp.float32),
            pltpu.VMEM((CE, D), jnp.float32),
            pltpu.VMEM_SHARED((n_acc, D), jnp.float32),
            pltpu.SemaphoreType.DMA,
            pltpu.SemaphoreType.DMA,
            pltpu.SemaphoreType.DMA,
            pltpu.SemaphoreType.DMA,
        ],
    )
    def agg(x_hbm, src_hbm, tgt_hbm, zero_hbm, part_hbm,
            src_v, tgt_v, buf0, buf1, buf2, buf3, acc,
            sem0, sem1, sem2, sem3):
        c = lax.axis_index("c")
        s = lax.axis_index("s")
        w = c * NS + s
        bufs = (buf0, buf1, buf2, buf3)
        sems = (sem0, sem1, sem2, sem3)
        r0 = s * rows_per
        pltpu.sync_copy(zero_hbm.at[pl.ds(r0, rows_per)],
                        acc.at[pl.ds(r0, rows_per)])
        plsc.subcore_barrier()

        @pl.loop(0, NCH // BLKC)
        def _(ib):
            hb = (w * NCH + ib * BLKC) * 2
            pltpu.sync_copy(src_hbm.at[pl.ds(hb, B2)], src_v)
            pltpu.sync_copy(tgt_hbm.at[pl.ds(hb, B2)], tgt_v)
            for bi in range(NSLOT):
                pltpu.async_copy(x_hbm.at[src_v.at[bi]], bufs[bi], sems[bi])

            @pl.loop(0, B2 // NSLOT - 1)
            def _(i):
                for bi in range(NSLOT):
                    k = i * NSLOT + bi
                    pltpu.make_async_copy(x_hbm.at[src_v.at[k]], bufs[bi],
                                          sems[bi]).wait()
                    pltpu.sync_copy(bufs[bi], acc.at[tgt_v.at[k]], add=True)
                    pltpu.async_copy(x_hbm.at[src_v.at[k + NSLOT]],
                                     bufs[bi], sems[bi])

            for bi in range(NSLOT):
                k = B2 - NSLOT + bi
                pltpu.make_async_copy(x_hbm.at[src_v.at[k]], bufs[bi],
                                      sems[bi]).wait()
                pltpu.sync_copy(bufs[bi], acc.at[tgt_v.at[k]], add=True)

        plsc.subcore_barrier()
        pltpu.sync_copy(acc.at[pl.ds(r0, rows_per)],
                        part_hbm.at[c].at[pl.ds(r0, rows_per)])

    return agg


def _dot_call(N, D, NCH, NP):
    CE = CH // 2
    B2 = 2 * BLKC
    NB = NCH // BLKC
    rows_stage = NP // NS
    mesh = plsc.VectorSubcoreMesh(core_axis_name="c", subcore_axis_name="s")

    @functools.partial(
        pl.kernel,
        out_type=jax.ShapeDtypeStruct((NW * NCH * CH * L,), jnp.float32),
        mesh=mesh,
        scratch_types=[
            pltpu.VMEM((BLKC, CH), jnp.int32),
            pltpu.VMEM((BLKC, CH), jnp.int32),
            pltpu.VMEM_SHARED((NP, D), jnp.float32),
            pltpu.VMEM((CE, D), jnp.float32),
            pltpu.VMEM((CE, D), jnp.float32),
            pltpu.VMEM((CE, D), jnp.float32),
            pltpu.VMEM((CE, D), jnp.float32),
            pltpu.VMEM((CE * L,), jnp.float32),
            pltpu.VMEM((CE * L,), jnp.float32),
            pltpu.SemaphoreType.DMA,
            pltpu.SemaphoreType.DMA,
            pltpu.SemaphoreType.DMA,
            pltpu.SemaphoreType.DMA,
        ],
    )
    def dot(h_hbm, src_hbm, tgt_hbm, out_hbm,
            src_v, tgt_v, hsp, bs0, bt0, bs1, bt1, pa0, pa1,
            sem0, sem1, semo0, semo1):
        c = lax.axis_index("c")
        s = lax.axis_index("s")
        w = c * NS + s
        bs, bt = (bs0, bs1), (bt0, bt1)
        pa = (pa0, pa1)
        sems = (sem0, sem1)
        semo = (semo0, semo1)

        r0 = s * rows_stage
        pltpu.sync_copy(h_hbm.at[pl.ds(r0, rows_stage)],
                        hsp.at[pl.ds(r0, rows_stage)])
        plsc.subcore_barrier()

        def idx_ref(v, hh):
            return v.at[hh // 2, pl.ds((hh % 2) * CE, CE)]

        def out_ref(ib, hh):
            off = (w * NCH * CH + ib * BLKC * CH + hh * CE) * L
            return out_hbm.at[pl.ds(off, CE * L)]

        def fire(hh, bi):
            pltpu.async_copy(hsp.at[idx_ref(src_v, hh)], bs[bi], sems[bi])
            pltpu.async_copy(hsp.at[idx_ref(tgt_v, hh)], bt[bi], sems[bi])

        def compute(hh, bi):
            pltpu.make_async_copy(hsp.at[idx_ref(src_v, hh)], bs[bi],
                                  sems[bi]).wait()
            pltpu.make_async_copy(hsp.at[idx_ref(tgt_v, hh)], bt[bi],
                                  sems[bi]).wait()

            @pl.loop(0, CE, unroll=4)
            def _(e):
                acc = bs[bi][e, pl.ds(0, L)] * bt[bi][e, pl.ds(0, L)]
                for k in range(1, D // L):
                    acc = acc + (bs[bi][e, pl.ds(k * L, L)] *
                                 bt[bi][e, pl.ds(k * L, L)])
                pa[bi][pl.ds(e * L, L)] = acc

        @pl.loop(0, NB)
        def _(ib):
            b0 = (w * NCH) + ib * BLKC
            pltpu.sync_copy(src_hbm.at[pl.ds(b0, BLKC)], src_v)
            pltpu.sync_copy(tgt_hbm.at[pl.ds(b0, BLKC)], tgt_v)
            for bi in range(2):
                fire(bi, bi)
            for bi in range(2):
                compute(bi, bi)
                fire(bi + 2, bi)
                pltpu.async_copy(pa[bi], out_ref(ib, bi), semo[bi])

            @pl.loop(1, B2 // 2 - 1)
            def _(i):
                for bi in range(2):
                    hh = i * 2 + bi
                    pltpu.make_async_copy(pa[bi], out_ref(ib, hh),
                                          semo[bi]).wait()
                    compute(hh, bi)
                    fire(hh + 2, bi)
                    pltpu.async_copy(pa[bi], out_ref(ib, hh), semo[bi])

            for bi in range(2):
                hh = B2 - 2 + bi
                pltpu.make_async_copy(pa[bi], out_ref(ib, hh),
                                      semo[bi]).wait()
                compute(hh, bi)
                pltpu.async_copy(pa[bi], out_ref(ib, hh), semo[bi])
            for bi in range(2):
                pltpu.make_async_copy(pa[bi], out_ref(ib, 0),
                                      semo[bi]).wait()

    return dot


def _reduce16(p):
    M = p.shape[0] // 128
    BLK = 4096
    EPR = 128 // L

    def red(pr, outr):
        x = pr[...]
        cols = [jnp.sum(x[:, e * L:(e + 1) * L], axis=1, keepdims=True)
                for e in range(EPR)]
        outr[...] = jnp.concatenate(cols, axis=1)

    return pl.pallas_call(
        red,
        grid=(M // BLK,),
        in_specs=[pl.BlockSpec((BLK, 128), lambda i: (i, 0))],
        out_specs=pl.BlockSpec((BLK, EPR), lambda i: (i, 0)),
        out_shape=jax.ShapeDtypeStruct((M, EPR), jnp.float32),
    )(p.reshape(M, 128))


def _dense(p0, p1, x, W_neigh, W_self, b2):
    N, D = x.shape
    BLK = 2000

    def mm(p0r, p1r, xr, wn, ws, br, hr):
        agg = p0r[...] + p1r[...]
        acc = jnp.dot(agg, wn[...], preferred_element_type=jnp.float32)
        acc = acc + jnp.dot(xr[...], ws[...], preferred_element_type=jnp.float32)
        hr[...] = jnp.maximum(acc + br[...], 0.0)

    row_spec = pl.BlockSpec((BLK, D), lambda i: (i, 0))
    w_spec = pl.BlockSpec((D, D), lambda i: (0, 0))
    return pl.pallas_call(
        mm,
        grid=(N // BLK,),
        in_specs=[row_spec, row_spec, row_spec, w_spec, w_spec,
                  pl.BlockSpec((1, D), lambda i: (0, 0))],
        out_specs=row_spec,
        out_shape=jax.ShapeDtypeStruct((N, D), jnp.float32),
    )(p0, p1, x, W_neigh, W_self, b2)


def kernel(x, edge_index, W_neigh, W_self, b):
    N, D = x.shape
    E = edge_index.shape[1]
    n_acc = -(-(N + 1) // 128) * 128
    NCH = -(-E // (CH * NW))
    NCH = -(-NCH // BLKC) * BLKC
    e_pad = NW * NCH * CH
    pad = e_pad - E

    src = edge_index[0]
    tgt = edge_index[1]
    src_p = jnp.concatenate(
        [src, jnp.zeros((pad,), jnp.int32)]).reshape(NW * NCH, CH)
    tgt_a = jnp.concatenate(
        [tgt, jnp.full((pad,), N, jnp.int32)]).reshape(NW * NCH, CH)
    tgt_c = jnp.concatenate(
        [tgt, jnp.zeros((pad,), jnp.int32)]).reshape(NW * NCH, CH)
    zeros = jnp.zeros((n_acc, D), jnp.float32)

    parts = _agg_call(N, D, NCH)(x, src_p.reshape(-1, CH // 2),
                                 tgt_a.reshape(-1, CH // 2), zeros)
    h = _dense(parts[0, :N], parts[1, :N], x, W_neigh, W_self,
               b.reshape(1, D))
    h_pad = jnp.concatenate([h, jnp.zeros((n_acc - N, D), jnp.float32)])
    partial16 = _dot_call(N, D, NCH, n_acc)(h_pad, src_p, tgt_c)
    scores = _reduce16(partial16)
    return scores.reshape(-1)[:E]
